# baseline (speedup 1.0000x reference)
import dataclasses
import functools

import jax
import jax.numpy as jnp
from jax import lax
from jax.experimental import pallas as pl
from jax.experimental.pallas import tpu as pltpu
from jax.experimental.pallas import tpu_sc as plsc

NC = 2
NS = 16
L = 16

EBLK = 512
SUB = 64
QCAP = 2 * SUB
ZDEN = 1000
BM = 1000


def _scal_cols(h, ap_ref, bm):
    sd = jnp.dot(h, ap_ref[...], preferred_element_type=jnp.float32)
    asrc = sd[:, 0]
    adst = sd[:, 1]
    cols = jnp.concatenate(
        [jnp.exp(asrc)[:, None], jnp.exp(0.2 * asrc)[:, None],
         jnp.exp(adst)[:, None], jnp.exp(0.2 * adst)[:, None],
         jnp.zeros((bm, 124), jnp.float32)], axis=1)
    return cols


def _head_body(bm, x_ref, w_ref, ap_ref, g_ref, scal_ref):
    h = jnp.dot(x_ref[...], w_ref[...], preferred_element_type=jnp.float32)
    cols = _scal_cols(h, ap_ref, bm)
    g_ref[0] = h * cols[:, 0][:, None]
    g_ref[1] = h * cols[:, 1][:, None]
    scal_ref[...] = cols


def _tc_head(x, w, apad, bm):
    n, d = x.shape
    return pl.pallas_call(
        functools.partial(_head_body, bm),
        grid=(n // bm,),
        in_specs=[
            pl.BlockSpec((bm, d), lambda i: (i, 0)),
            pl.BlockSpec((d, d), lambda i: (0, 0)),
            pl.BlockSpec((d, d), lambda i: (0, 0)),
        ],
        out_specs=[
            pl.BlockSpec((2, bm, d), lambda i: (0, i, 0)),
            pl.BlockSpec((bm, d), lambda i: (i, 0)),
        ],
        out_shape=[
            jax.ShapeDtypeStruct((2, n, d), jnp.float32),
            jax.ShapeDtypeStruct((n, d), jnp.float32),
        ],
    )(x, w, apad)


def _combine(acc_ref, dent_ref, scal_ref):
    pd = scal_ref[:, 2]
    pd2 = scal_ref[:, 3]
    num = pd[:, None] * acc_ref[0] + pd2[:, None] * acc_ref[1]
    den = pd * dent_ref[:, 0] + pd2 * dent_ref[:, 1] + 1e-16
    return num / den[:, None]


def _mid_body(bm, acc_ref, dent_ref, scal_ref, w_ref, ap_ref, g_ref, scal2_ref):
    out1 = _combine(acc_ref, dent_ref, scal_ref)
    h = jnp.dot(out1, w_ref[...], preferred_element_type=jnp.float32)
    cols = _scal_cols(h, ap_ref, bm)
    g_ref[0] = h * cols[:, 0][:, None]
    g_ref[1] = h * cols[:, 1][:, None]
    scal2_ref[...] = cols


def _tc_mid(acc, dent, scal, w, apad, bm):
    _, n, d = acc.shape
    return pl.pallas_call(
        functools.partial(_mid_body, bm),
        grid=(n // bm,),
        in_specs=[
            pl.BlockSpec((2, bm, d), lambda i: (0, i, 0)),
            pl.BlockSpec((bm, 2), lambda i: (i, 0)),
            pl.BlockSpec((bm, d), lambda i: (i, 0)),
            pl.BlockSpec((d, d), lambda i: (0, 0)),
            pl.BlockSpec((d, d), lambda i: (0, 0)),
        ],
        out_specs=[
            pl.BlockSpec((2, bm, d), lambda i: (0, i, 0)),
            pl.BlockSpec((bm, d), lambda i: (i, 0)),
        ],
        out_shape=[
            jax.ShapeDtypeStruct((2, n, d), jnp.float32),
            jax.ShapeDtypeStruct((n, d), jnp.float32),
        ],
    )(acc, dent, scal, w, apad)


def _final_body(acc_ref, dent_ref, scal_ref, o_ref):
    o_ref[...] = _combine(acc_ref, dent_ref, scal_ref)


def _tc_final(acc, dent, scal, bm):
    _, n, d = acc.shape
    return pl.pallas_call(
        _final_body,
        grid=(n // bm,),
        in_specs=[
            pl.BlockSpec((2, bm, d), lambda i: (0, i, 0)),
            pl.BlockSpec((bm, 2), lambda i: (i, 0)),
            pl.BlockSpec((bm, d), lambda i: (i, 0)),
        ],
        out_specs=pl.BlockSpec((bm, d), lambda i: (i, 0)),
        out_shape=jax.ShapeDtypeStruct((n, d), jnp.float32),
    )(acc, dent, scal)


def _sc_edge_pass(g, p, q, pd, src, dst):
    n = p.shape[0]
    d = g.shape[1]
    e = src.shape[0]
    nblk = e // EBLK
    itersb = pl.cdiv(nblk, NS)
    nacc = n + 8
    nden = ((n + 1 + ZDEN - 1) // ZDEN) * ZDEN
    trash = n

    mesh = plsc.VectorSubcoreMesh(core_axis_name="c", subcore_axis_name="s")

    cp = pltpu.CompilerParams()
    if "needs_layout_passes" in pltpu.CompilerParams.__dataclass_fields__:
        cp = dataclasses.replace(cp, needs_layout_passes=False)

    @functools.partial(
        pl.kernel,
        compiler_params=cp,
        out_type=[
            jax.ShapeDtypeStruct((NC, n, d), jnp.float32),
            jax.ShapeDtypeStruct((NC, 1, nden), jnp.float32),
        ],
        mesh=mesh,
        scratch_types=[
            pltpu.VMEM_SHARED((nacc, d), jnp.float32),
            pltpu.VMEM_SHARED((nden,), jnp.float32),
            pltpu.VMEM((n,), jnp.float32),
            pltpu.VMEM((n,), jnp.float32),
            pltpu.VMEM((n,), jnp.float32),
            pltpu.VMEM((EBLK,), jnp.int32),
            pltpu.VMEM((EBLK,), jnp.int32),
            pltpu.VMEM((QCAP,), jnp.int32),
            pltpu.VMEM((QCAP,), jnp.int32),
            pltpu.VMEM((QCAP,), jnp.float32),
            pltpu.VMEM((SUB,), jnp.int32),
            pltpu.VMEM((SUB, 128), jnp.float32),
            pltpu.VMEM((ZDEN,), jnp.float32),
            pltpu.SemaphoreType.DMA,
        ],
    )
    def edge_kernel(g_hbm, p_hbm, q_hbm, pd_hbm, src_hbm, dst_hbm,
                    acc_out, den_out,
                    acc_sh, den_sh, p_v, q_v, pd_v, sbig_v, dbig_v,
                    qg_v, qd_v, qv_v, qdf_v, rows_v, zden_v, gsem):
        cid = lax.axis_index("c")
        sid = lax.axis_index("s")
        me_pos = cid == 0

        zero16 = jnp.zeros((L,), jnp.float32)

        @pl.loop(0, SUB)
        def _(r):
            for j in range(d // L):
                rows_v[r, pl.ds(j * L, L)] = zero16

        nzb = nacc // SUB
        ztail = nacc - nzb * SUB

        @pl.loop(0, pl.cdiv(nzb, NS))
        def _(t):
            k = t * NS + sid

            @pl.when(k < nzb)
            def _():
                pltpu.sync_copy(rows_v, acc_sh.at[pl.ds(k * SUB, SUB)])

        if ztail:
            @pl.when(sid == NS - 1)
            def _():
                pltpu.sync_copy(rows_v.at[pl.ds(0, ztail)],
                                acc_sh.at[pl.ds(nzb * SUB, ztail)])

        @pl.when(sid == 0)
        def _():
            @pl.loop(0, ZDEN // L)
            def _(k):
                zden_v[pl.ds(k * L, L)] = zero16

            @pl.loop(0, nden // ZDEN)
            def _(k):
                pltpu.sync_copy(zden_v, den_sh.at[pl.ds(k * ZDEN, ZDEN)])

        pltpu.sync_copy(p_hbm, p_v)
        pltpu.sync_copy(q_hbm, q_v)
        pltpu.sync_copy(pd_hbm, pd_v)

        plsc.subcore_barrier()

        def flush():
            pltpu.make_async_copy(g_hbm.at[qg_v.at[pl.ds(0, SUB)]],
                                  rows_v, gsem).start()
            for k in range(SUB // L):
                qdf_v[pl.ds(k * L, L)] = qd_v[pl.ds(k * L, L)]
            pltpu.sync_copy(qv_v.at[pl.ds(0, SUB)],
                            den_sh.at[qdf_v], add=True)
            pltpu.make_async_copy(g_hbm.at[qg_v.at[pl.ds(0, SUB)]],
                                  rows_v, gsem).wait()
            pltpu.sync_copy(rows_v, acc_sh.at[qdf_v], add=True)
            for k in range(SUB // L):
                qg_v[pl.ds(k * L, L)] = qg_v[pl.ds(SUB + k * L, L)]
                qd_v[pl.ds(k * L, L)] = qd_v[pl.ds(SUB + k * L, L)]
                qv_v[pl.ds(k * L, L)] = qv_v[pl.ds(SUB + k * L, L)]

        def sub_body(valid, s, qc):
            for grp in range(SUB // L):
                off = s * SUB + grp * L
                si = sbig_v[pl.ds(off, L)]
                di = dbig_v[pl.ds(off, L)]
                pu = plsc.load_gather(p_v, [si])
                qu = plsc.load_gather(q_v, [si])
                pv = plsc.load_gather(pd_v, [di])
                pos = (pu * pv) >= 1.0
                mask = jnp.logical_and(jnp.equal(pos, me_pos), valid)
                dval = jnp.where(me_pos, pu, qu)
                gidx = si + cid * n
                m32 = mask.astype(jnp.int32)
                cs = plsc.cumsum(m32)
                slot = qc + cs - m32
                plsc.store_scatter(qg_v, [slot], gidx, mask=mask)
                plsc.store_scatter(qd_v, [slot], di, mask=mask)
                plsc.store_scatter(qv_v, [slot], dval, mask=mask)
                qc = qc + plsc.all_reduce_population_count(mask)
            nq = jnp.max(qc)
            do_f = nq >= SUB

            @pl.when(do_f)
            def _():
                flush()

            return jnp.where(do_f, qc - SUB, qc)

        def blk_body(it, qc):
            blk = it * NS + sid
            valid = blk < nblk

            @pl.when(valid)
            def _():
                base = blk * EBLK
                pltpu.sync_copy(src_hbm.at[pl.ds(base, EBLK)], sbig_v)
                pltpu.sync_copy(dst_hbm.at[pl.ds(base, EBLK)], dbig_v)

            qc = pl.loop(0, EBLK // SUB, init_carry=qc)(
                functools.partial(sub_body, valid))
            return qc

        qcf = pl.loop(0, itersb, init_carry=jnp.zeros((L,), jnp.int32))(blk_body)

        lanes = lax.iota(jnp.int32, L)
        for grp in range(QCAP // L):
            pos_l = lanes + grp * L
            padmask = pos_l >= qcf
            plsc.store_scatter(qg_v, [pos_l],
                               jnp.zeros((L,), jnp.int32), mask=padmask)
            plsc.store_scatter(qd_v, [pos_l],
                               jnp.full((L,), trash, jnp.int32), mask=padmask)
            plsc.store_scatter(qv_v, [pos_l],
                               jnp.zeros((L,), jnp.float32), mask=padmask)
        flush()
        flush()

        plsc.subcore_barrier()

        crows = (n // NS) // 8 * 8
        tail = n - NS * crows
        rbase = sid * crows
        pltpu.sync_copy(acc_sh.at[pl.ds(rbase, crows)],
                        acc_out.at[cid].at[pl.ds(rbase, crows)])

        @pl.when(sid == 0)
        def _():
            if tail:
                pltpu.sync_copy(acc_sh.at[pl.ds(NS * crows, tail)],
                                acc_out.at[cid].at[pl.ds(NS * crows, tail)])
            pltpu.sync_copy(den_sh, den_out.at[cid].at[0])

    return edge_kernel(g, p, q, pd, src, dst)


def kernel(x, edges, W1, a1_src, a1_dst, W2, a2_src, a2_dst):
    n, d = x.shape
    src = edges[0].astype(jnp.int32)
    dst = edges[1].astype(jnp.int32)

    ap1 = jnp.zeros((d, d), jnp.float32).at[:, 0].set(a1_src).at[:, 1].set(a1_dst)
    ap2 = jnp.zeros((d, d), jnp.float32).at[:, 0].set(a2_src).at[:, 1].set(a2_dst)

    g1, scal1 = _tc_head(x, W1, ap1, BM)
    acc1, den1 = _sc_edge_pass(g1.reshape(2 * n, d), scal1[:, 0], scal1[:, 1],
                               scal1[:, 2], src, dst)
    g2, scal2 = _tc_mid(acc1, den1[:, 0, :n].swapaxes(0, 1), scal1, W2, ap2, BM)
    acc2, den2 = _sc_edge_pass(g2.reshape(2 * n, d), scal2[:, 0], scal2[:, 1],
                               scal2[:, 2], src, dst)
    return _tc_final(acc2, den2[:, 0, :n].swapaxes(0, 1), scal2, BM)
